# baseline (speedup 1.0000x reference)
import jax
import jax.numpy as jnp
from jax import lax
from jax.experimental import pallas as pl
from jax.experimental.pallas import tpu as pltpu, tpu_sc as plsc

_NC, _NS, _L = 1, 16, 16
_NW = _NC * _NS
_N = 100000
_CHUNK = 6272


def _body(e_hbm, s_hbm, v_hbm, out_hbm, e_v, s_v, tab_v, sem_e, sem_s, sem_t):
    wid = lax.axis_index("s") * _NC + lax.axis_index("c")
    base = jnp.minimum(wid * _CHUNK, _N - _CHUNK)
    ct = pltpu.async_copy(v_hbm, tab_v, sem_t)
    ce = pltpu.async_copy(
        e_hbm.at[pl.ds(base, _CHUNK)], e_v, sem_e)
    cs = pltpu.async_copy(
        s_hbm.at[pl.ds(base, _CHUNK)], s_v, sem_s)
    ct.wait()
    ce.wait()
    cs.wait()

    @plsc.parallel_loop(0, _CHUNK, step=_L, unroll=4)
    def _step(i):
        sl = pl.ds(i, _L)
        vals = plsc.load_gather(tab_v, [s_v[sl]])
        plsc.addupdate(e_v.at[sl], vals)

    pltpu.sync_copy(e_v, out_hbm.at[pl.ds(base, _CHUNK)])


@jax.jit
def _sc_rescale(e, s, v):
    mesh = plsc.VectorSubcoreMesh(
        core_axis_name="c", subcore_axis_name="s", num_cores=_NC)
    return pl.kernel(
        _body,
        out_type=jax.ShapeDtypeStruct((_N,), jnp.float32),
        mesh=mesh,
        scratch_types=[
            pltpu.VMEM((_CHUNK,), jnp.float32),
            pltpu.VMEM((_CHUNK,), jnp.int32),
            pltpu.VMEM((120,), jnp.float32),
            pltpu.SemaphoreType.DMA,
            pltpu.SemaphoreType.DMA,
            pltpu.SemaphoreType.DMA,
        ],
        compiler_params=pltpu.CompilerParams(
            needs_layout_passes=False,
            disable_bounds_checks=True,
            disable_semaphore_checks=True,
            skip_device_barrier=True,
        ),
    )(e, s, v)


def kernel(energies, node_species, values):
    return _sc_rescale(energies, node_species, values)


# speedup vs baseline: 1.0053x; 1.0053x over previous
import jax
import jax.numpy as jnp
from jax import lax
from jax.experimental import pallas as pl
from jax.experimental.pallas import tpu as pltpu, tpu_sc as plsc

_NC, _NS, _L = 1, 16, 16
_NW = _NC * _NS
_N = 100000
_CHUNK = 6272


def _body(e_hbm, s_hbm, v_hbm, out_hbm, e_v, s_v, tab_v, sem_e, sem_s, sem_t):
    wid = lax.axis_index("s") * _NC + lax.axis_index("c")
    base = jnp.minimum(wid * _CHUNK, _N - _CHUNK)
    ct = pltpu.async_copy(v_hbm, tab_v, sem_t)
    ce = pltpu.async_copy(
        e_hbm.at[pl.ds(base, _CHUNK)], e_v, sem_e)
    cs = pltpu.async_copy(
        s_hbm.at[pl.ds(base, _CHUNK)], s_v, sem_s)
    ct.wait()
    ce.wait()
    cs.wait()

    @plsc.parallel_loop(0, _CHUNK, step=_L, unroll=8)
    def _step(i):
        sl = pl.ds(i, _L)
        vals = plsc.load_gather(tab_v, [s_v[sl]])
        plsc.addupdate(e_v.at[sl], vals)

    pltpu.sync_copy(e_v, out_hbm.at[pl.ds(base, _CHUNK)])


@jax.jit
def _sc_rescale(e, s, v):
    mesh = plsc.VectorSubcoreMesh(
        core_axis_name="c", subcore_axis_name="s", num_cores=_NC)
    return pl.kernel(
        _body,
        out_type=jax.ShapeDtypeStruct((_N,), jnp.float32),
        mesh=mesh,
        scratch_types=[
            pltpu.VMEM((_CHUNK,), jnp.float32),
            pltpu.VMEM((_CHUNK,), jnp.int32),
            pltpu.VMEM((120,), jnp.float32),
            pltpu.SemaphoreType.DMA,
            pltpu.SemaphoreType.DMA,
            pltpu.SemaphoreType.DMA,
        ],
        compiler_params=pltpu.CompilerParams(
            needs_layout_passes=False,
            disable_bounds_checks=True,
            disable_semaphore_checks=True,
            skip_device_barrier=True,
        ),
    )(e, s, v)


def kernel(energies, node_species, values):
    return _sc_rescale(energies, node_species, values)
